# baseline (speedup 1.0000x reference)
import functools

import jax
import jax.numpy as jnp
from jax.experimental import pallas as pl

_H = 16


def _mm_bias_body(x_ref, w_ref, b_ref, o_ref):
    x = x_ref[...].astype(jnp.bfloat16)
    acc = jax.lax.dot(x, w_ref[...], preferred_element_type=jnp.float32)
    o_ref[...] = acc + b_ref[...]


def _mm_bias(x, w, b, bm=512):
    M, K = x.shape
    N = w.shape[1]
    wbf = w.astype(jnp.bfloat16)
    b2 = b.reshape(1, N)
    return pl.pallas_call(
        _mm_bias_body,
        grid=(M // bm,),
        in_specs=[
            pl.BlockSpec((bm, K), lambda i: (i, 0)),
            pl.BlockSpec((K, N), lambda i: (0, 0)),
            pl.BlockSpec((1, N), lambda i: (0, 0)),
        ],
        out_specs=pl.BlockSpec((bm, N), lambda i: (i, 0)),
        out_shape=jax.ShapeDtypeStruct((M, N), jnp.float32),
    )(x, wbf, b2)


def _attn_body(q_ref, k_ref, v_ref, o_ref, *, scale):
    q = q_ref[0, 0].astype(jnp.bfloat16)
    k = k_ref[0, 0].astype(jnp.bfloat16)
    v = v_ref[0, 0].astype(jnp.bfloat16)
    s = jax.lax.dot_general(
        q, k, (((1,), (1,)), ((), ())),
        preferred_element_type=jnp.float32) * scale
    m = jnp.max(s, axis=-1, keepdims=True)
    e = jnp.exp(s - m)
    p = e / jnp.sum(e, axis=-1, keepdims=True)
    o_ref[0, 0] = jax.lax.dot(
        p.astype(jnp.bfloat16), v, preferred_element_type=jnp.float32)


def _attention(qh, kh, vh, bq=512):
    B, H, S, hd = qh.shape
    bq = min(bq, S)
    scale = 1.0 / (hd ** 0.5)
    return pl.pallas_call(
        functools.partial(_attn_body, scale=scale),
        grid=(B, H, S // bq),
        in_specs=[
            pl.BlockSpec((1, 1, bq, hd), lambda b, h, i: (b, h, i, 0)),
            pl.BlockSpec((1, 1, S, hd), lambda b, h, i: (b, h, 0, 0)),
            pl.BlockSpec((1, 1, S, hd), lambda b, h, i: (b, h, 0, 0)),
        ],
        out_specs=pl.BlockSpec((1, 1, bq, hd), lambda b, h, i: (b, h, i, 0)),
        out_shape=jax.ShapeDtypeStruct((B, H, S, hd), jnp.float32),
    )(qh, kh, vh)


def _oproj_ln_body(a_ref, w_ref, b_ref, r_ref, g_ref, be_ref, o_ref):
    a = a_ref[...].astype(jnp.bfloat16)
    y = jax.lax.dot(a, w_ref[...], preferred_element_type=jnp.float32)
    x = r_ref[...] + y + b_ref[...]
    m = jnp.mean(x, axis=-1, keepdims=True)
    var = jnp.mean((x - m) ** 2, axis=-1, keepdims=True)
    o_ref[...] = (x - m) * jax.lax.rsqrt(var + 1e-5) * g_ref[...] + be_ref[...]


def _oproj_ln(a, w, b, resid, g, beta, bm=512):
    M, K = a.shape
    N = w.shape[1]
    wbf = w.astype(jnp.bfloat16)
    return pl.pallas_call(
        _oproj_ln_body,
        grid=(M // bm,),
        in_specs=[
            pl.BlockSpec((bm, K), lambda i: (i, 0)),
            pl.BlockSpec((K, N), lambda i: (0, 0)),
            pl.BlockSpec((1, N), lambda i: (0, 0)),
            pl.BlockSpec((bm, N), lambda i: (i, 0)),
            pl.BlockSpec((1, N), lambda i: (0, 0)),
            pl.BlockSpec((1, N), lambda i: (0, 0)),
        ],
        out_specs=pl.BlockSpec((bm, N), lambda i: (i, 0)),
        out_shape=jax.ShapeDtypeStruct((M, N), jnp.float32),
    )(a, wbf, b.reshape(1, N), resid, g.reshape(1, N), beta.reshape(1, N))


def _router_body(x_ref, wg_ref, cmb_ref, f_ref, p_ref, z_ref, aux_ref,
                 *, nsteps, T, E):
    i = pl.program_id(0)
    x = x_ref[...].astype(jnp.bfloat16)
    logits = jax.lax.dot(x, wg_ref[...], preferred_element_type=jnp.float32)
    mx = jnp.max(logits, axis=-1, keepdims=True)
    ex = jnp.exp(logits - mx)
    se = jnp.sum(ex, axis=-1, keepdims=True)
    probs = ex / se

    iota = jax.lax.broadcasted_iota(jnp.int32, probs.shape, 1)
    v1 = jnp.max(probs, axis=-1, keepdims=True)
    i1 = jnp.min(jnp.where(probs == v1, iota, E), axis=-1, keepdims=True)
    masked = jnp.where(iota == i1, -jnp.inf, probs)
    v2 = jnp.max(masked, axis=-1, keepdims=True)
    i2 = jnp.min(jnp.where(masked == v2, iota, E), axis=-1, keepdims=True)

    combine = (jnp.where(iota == i1, v1, 0.0)
               + jnp.where(iota == i2, v2, 0.0))
    cmb_ref[...] = combine

    f_part = jnp.sum(jnp.where(iota == i1, 1.0, 0.0), axis=0, keepdims=True)
    p_part = jnp.sum(probs, axis=0, keepdims=True)
    lse = mx + jnp.log(se)
    z_part = jnp.sum(lse * lse).reshape(1, 1)

    @pl.when(i == 0)
    def _():
        f_ref[...] = f_part
        p_ref[...] = p_part
        z_ref[...] = z_part

    @pl.when(i > 0)
    def _():
        f_ref[...] += f_part
        p_ref[...] += p_part
        z_ref[...] += z_part

    @pl.when(i == nsteps - 1)
    def _():
        invT = 1.0 / T
        bal = E * jnp.sum(f_ref[...] * invT * (p_ref[...] * invT))
        aux_ref[...] = (bal * 1e-2 + z_ref[0, 0] * invT * 1e-3).reshape(1, 1)


def _router(x, wg, bm=512):
    T, D = x.shape
    E = wg.shape[1]
    nsteps = T // bm
    return pl.pallas_call(
        functools.partial(_router_body, nsteps=nsteps, T=T, E=E),
        grid=(nsteps,),
        in_specs=[
            pl.BlockSpec((bm, D), lambda i: (i, 0)),
            pl.BlockSpec((D, E), lambda i: (0, 0)),
        ],
        out_specs=[
            pl.BlockSpec((bm, E), lambda i: (i, 0)),
            pl.BlockSpec((1, E), lambda i: (0, 0)),
            pl.BlockSpec((1, E), lambda i: (0, 0)),
            pl.BlockSpec((1, 1), lambda i: (0, 0)),
            pl.BlockSpec((1, 1), lambda i: (0, 0)),
        ],
        out_shape=[
            jax.ShapeDtypeStruct((T, E), jnp.float32),
            jax.ShapeDtypeStruct((1, E), jnp.float32),
            jax.ShapeDtypeStruct((1, E), jnp.float32),
            jax.ShapeDtypeStruct((1, 1), jnp.float32),
            jax.ShapeDtypeStruct((1, 1), jnp.float32),
        ],
    )(x, wg.astype(jnp.bfloat16))


def _moe_body(x_ref, w1_ref, w2_ref, cmb_ref, g_ref, be_ref, o_ref, *, E):
    e = pl.program_id(1)
    x = x_ref[...]
    xb = x.astype(jnp.bfloat16)
    h = jax.lax.dot(xb, w1_ref[0], preferred_element_type=jnp.float32)
    h = jax.nn.gelu(h)
    eo = jax.lax.dot(h.astype(jnp.bfloat16), w2_ref[0],
                     preferred_element_type=jnp.float32)
    iota = jax.lax.broadcasted_iota(jnp.int32, cmb_ref.shape, 1)
    w = jnp.sum(jnp.where(iota == e, cmb_ref[...], 0.0), axis=1, keepdims=True)
    contrib = w * eo

    @pl.when(e == 0)
    def _():
        o_ref[...] = contrib

    @pl.when(e > 0)
    def _():
        o_ref[...] += contrib

    @pl.when(e == E - 1)
    def _():
        t = x + o_ref[...]
        m = jnp.mean(t, axis=-1, keepdims=True)
        var = jnp.mean((t - m) ** 2, axis=-1, keepdims=True)
        o_ref[...] = (t - m) * jax.lax.rsqrt(var + 1e-5) * g_ref[...] + be_ref[...]


def _moe(x, w1, w2, cmb, g, beta, bm=512):
    T, D = x.shape
    E, _, FF = w1.shape
    return pl.pallas_call(
        functools.partial(_moe_body, E=E),
        grid=(T // bm, E),
        in_specs=[
            pl.BlockSpec((bm, D), lambda i, e: (i, 0)),
            pl.BlockSpec((1, D, FF), lambda i, e: (e, 0, 0)),
            pl.BlockSpec((1, FF, D), lambda i, e: (e, 0, 0)),
            pl.BlockSpec((bm, E), lambda i, e: (i, 0)),
            pl.BlockSpec((1, D), lambda i, e: (0, 0)),
            pl.BlockSpec((1, D), lambda i, e: (0, 0)),
        ],
        out_specs=pl.BlockSpec((bm, D), lambda i, e: (i, 0)),
        out_shape=jax.ShapeDtypeStruct((T, D), jnp.float32),
    )(x, w1.astype(jnp.bfloat16), w2.astype(jnp.bfloat16), cmb,
      g.reshape(1, D), beta.reshape(1, D))


def kernel(q, k, v, Wq, bq, Wk, bk, Wv, bv, Wo, bo, ln1_g, ln1_b,
           Wg, W1, W2, ln2_g, ln2_b):
    B, S, D = q.shape
    H = _H
    hd = D // H
    T = B * S

    q2 = q.reshape(T, D)
    qp = _mm_bias(q2, Wq, bq)
    kp = _mm_bias(k.reshape(T, D), Wk, bk)
    vp = _mm_bias(v.reshape(T, D), Wv, bv)

    qh = qp.reshape(B, S, H, hd).transpose(0, 2, 1, 3)
    kh = kp.reshape(B, S, H, hd).transpose(0, 2, 1, 3)
    vh = vp.reshape(B, S, H, hd).transpose(0, 2, 1, 3)

    ao = _attention(qh, kh, vh)
    ao2 = ao.transpose(0, 2, 1, 3).reshape(T, D)

    x = _oproj_ln(ao2, Wo, bo, q2, ln1_g, ln1_b)

    cmb, _f, _p, _z, aux = _router(x, Wg)

    out = _moe(x, W1, W2, cmb, ln2_g, ln2_b)
    return out.reshape(B, S, D), aux[0, 0]
